# baseline (speedup 1.0000x reference)
import functools

import jax
import jax.numpy as jnp
from jax import lax
from jax.experimental import pallas as pl
from jax.experimental.pallas import tpu as pltpu
from jax.experimental.pallas import tpu_sc as plsc

B_, C_, NX, NY = 4, 96, 384, 384
N_ = 8192
NW = 32
CH = 32
L = 16
CP = 128

MAGIC = 12582912.0
OFFN = [1.5, 1.125, 0.75, 0.375, 0.0, -0.375, -0.75, -1.125, -1.5]
IOFF = [1.0, 0.0, -1.0]


def _rne(v):
    return (v + jnp.float32(MAGIC)) - jnp.float32(MAGIC)


def _axis_weights(r, p, hi):
    groups = []
    for j in range(3):
        acc = None
        for m in range(3):
            pp = jnp.clip(r - jnp.float32(OFFN[3 * j + m]), 0.0, hi)
            d = pp - p
            e = jnp.exp(jnp.float32(-2.0) * d * d)
            acc = e if acc is None else acc + e
        groups.append(acc)
    inv = jnp.float32(1.0) / (groups[0] + groups[1] + groups[2])
    return [g * inv for g in groups]


def _make_sc_kernel(nsamp):
    samp_w = nsamp // NW
    nchunk = samp_w // CH
    assert nchunk % 2 == 0 and nchunk >= 4
    mesh = plsc.VectorSubcoreMesh(core_axis_name="c", subcore_axis_name="s")

    @functools.partial(
        pl.kernel,
        mesh=mesh,
        compiler_params=pltpu.CompilerParams(needs_layout_passes=False),
        out_type=jax.ShapeDtypeStruct((nsamp, CP), jnp.float32),
        scratch_types=[
            pltpu.VMEM((samp_w,), jnp.float32),
            pltpu.VMEM((samp_w,), jnp.float32),
            pltpu.VMEM((2 * 9, CH), jnp.int32),
            pltpu.VMEM((2 * 9 * CH,), jnp.float32),
            pltpu.VMEM((2 * 9, CH, CP), jnp.float32),
            pltpu.VMEM((2 * CH, CP), jnp.float32),
            pltpu.SemaphoreType.DMA,
            pltpu.SemaphoreType.DMA,
            pltpu.SemaphoreType.DMA,
            pltpu.SemaphoreType.DMA,
        ],
    )
    def k(xt_hbm, px_hbm, py_hbm, out_hbm,
          px_v, py_v, idx_v, w_v, rows_v, out_v, gs0, gs1, os0, os1):
        gsem = [gs0, gs1]
        osem = [os0, os1]
        wid = lax.axis_index("s") * 2 + lax.axis_index("c")
        base = wid * samp_w
        pltpu.sync_copy(px_hbm.at[pl.ds(base, samp_w)], px_v)
        pltpu.sync_copy(py_hbm.at[pl.ds(base, samp_w)], py_v)
        nbat = nsamp // N_
        bterm = (wid // (NW // nbat)) * (NX * NY)

        def stage(g, b):
            l0 = g * CH
            for t in range(CH // L):
                o = l0 + t * L
                cx = px_v[pl.ds(o, L)] * jnp.float32(NX - 1)
                cy = py_v[pl.ds(o, L)] * jnp.float32(NY - 1)
                rx = _rne(cx)
                ry = _rne(cy)
                wx = _axis_weights(rx, cx, jnp.float32(NX))
                wy = _axis_weights(ry, cy, jnp.float32(NX))
                ix = [jnp.clip(rx + jnp.float32(ofs), 0.0, jnp.float32(NX - 1))
                      .astype(jnp.int32) for ofs in IOFF]
                iy = [jnp.clip(ry + jnp.float32(ofs), 0.0, jnp.float32(NY - 1))
                      .astype(jnp.int32) for ofs in IOFF]
                for j in range(3):
                    rowj = ix[j] * NY + bterm
                    for kk in range(3):
                        tap = j * 3 + kk
                        idx_v[b * 9 + tap, pl.ds(t * L, L)] = rowj + iy[kk]
                        w_v[pl.ds((b * 9 + tap) * CH + t * L, L)] = wx[j] * wy[kk]
            for tap in range(9):
                pltpu.async_copy(xt_hbm.at[idx_v.at[b * 9 + tap]],
                                 rows_v.at[b * 9 + tap], gsem[b])

        def wait_gathers(b):
            for tap in range(9):
                pltpu.make_async_copy(xt_hbm.at[idx_v.at[b * 9 + tap]],
                                      rows_v.at[b * 9 + tap], gsem[b]).wait()

        def accumulate(g, b):
            def acc_body(s, _carry):
                sv = jnp.full((L,), s, jnp.int32)
                wb = [plsc.load_gather(w_v, [sv + (b * 9 + tap) * CH])
                      for tap in range(9)]
                for c6 in range(C_ // L):
                    a = wb[0] * rows_v[b * 9, s, pl.ds(c6 * L, L)]
                    for tap in range(1, 9):
                        a = a + wb[tap] * rows_v[b * 9 + tap, s, pl.ds(c6 * L, L)]
                    out_v[b * CH + s, pl.ds(c6 * L, L)] = a
                return 0
            lax.fori_loop(0, CH, acc_body, 0)
            pltpu.async_copy(out_v.at[pl.ds(b * CH, CH)],
                             out_hbm.at[pl.ds(base + g * CH, CH)], osem[b])

        def wait_out(g, b):
            pltpu.make_async_copy(out_v.at[pl.ds(b * CH, CH)],
                                  out_hbm.at[pl.ds(base + g * CH, CH)],
                                  osem[b]).wait()

        stage(0, 0)
        stage(1, 1)

        def body(i, _):
            for b in (0, 1):
                g = 2 * i + b
                wait_gathers(b)

                @pl.when(i > 0)
                def _():
                    wait_out(g - 2, b)

                accumulate(g, b)
                stage(g + 2, b)
            return 0

        lax.fori_loop(0, nchunk // 2 - 1, body, 0)

        for b in (0, 1):
            g = nchunk - 2 + b
            wait_gathers(b)
            wait_out(g - 2, b)
            accumulate(g, b)
            wait_out(g, b)

    return k


_sc_sample = _make_sc_kernel(B_ * N_)

NPIX = NX * NY
RB = 64


def _tc_transpose_body(x_ref, o_ref):
    zpad = jnp.zeros((NY, CP - C_), jnp.float32)
    for rr in range(RB):
        t = jnp.transpose(x_ref[0, :, rr, :], (1, 0))
        o_ref[pl.ds(rr * NY, NY), :] = jnp.concatenate([t, zpad], axis=1)


_tc_transpose = pl.pallas_call(
    _tc_transpose_body,
    grid=(B_, NX // RB),
    in_specs=[pl.BlockSpec((1, C_, RB, NY), lambda b, r: (b, 0, r, 0))],
    out_specs=pl.BlockSpec((RB * NY, CP),
                           lambda b, r: (b * (NX // RB) + r, 0)),
    out_shape=jax.ShapeDtypeStruct((B_ * NPIX, CP), jnp.float32),
)


NB2 = 2048


def _tc_out_body(o_ref, y_ref):
    y_ref[...] = jnp.transpose(o_ref[:, :C_], (1, 0))[None]


_tc_out = pl.pallas_call(
    _tc_out_body,
    grid=(B_, N_ // NB2),
    in_specs=[pl.BlockSpec((NB2, CP),
                           lambda b, i: (b * (N_ // NB2) + i, 0))],
    out_specs=pl.BlockSpec((1, C_, NB2), lambda b, i: (b, 0, i)),
    out_shape=jax.ShapeDtypeStruct((B_, C_, N_), jnp.float32),
)


def kernel(x, coords):
    b, c, nx, ny = x.shape
    n = coords.shape[1]
    assert (b, c, nx, ny) == (B_, C_, NX, NY) and n == N_
    xt = _tc_transpose(x)
    px = coords[:, :, 1].reshape(-1)
    py = coords[:, :, 0].reshape(-1)
    out = _sc_sample(xt, px, py)
    return _tc_out(out)


# speedup vs baseline: 1.0109x; 1.0109x over previous
import functools

import jax
import jax.numpy as jnp
from jax import lax
from jax.experimental import pallas as pl
from jax.experimental.pallas import tpu as pltpu
from jax.experimental.pallas import tpu_sc as plsc

B_, C_, NX, NY = 4, 96, 384, 384
N_ = 8192
NW = 32
CH = 32
L = 16
CP = 128

MAGIC = 12582912.0
OFFN = [1.5, 1.125, 0.75, 0.375, 0.0, -0.375, -0.75, -1.125, -1.5]
IOFF = [1.0, 0.0, -1.0]


def _rne(v):
    return (v + jnp.float32(MAGIC)) - jnp.float32(MAGIC)


def _axis_weights(r, p, hi):
    groups = []
    for j in range(3):
        acc = None
        for m in range(3):
            pp = jnp.clip(r - jnp.float32(OFFN[3 * j + m]), 0.0, hi)
            d = pp - p
            e = jnp.exp(jnp.float32(-2.0) * d * d)
            acc = e if acc is None else acc + e
        groups.append(acc)
    inv = jnp.float32(1.0) / (groups[0] + groups[1] + groups[2])
    return [g * inv for g in groups]


def _make_sc_kernel(nsamp):
    samp_w = nsamp // NW
    nchunk = samp_w // CH
    assert nchunk % 2 == 0 and nchunk >= 4
    mesh = plsc.VectorSubcoreMesh(core_axis_name="c", subcore_axis_name="s")

    @functools.partial(
        pl.kernel,
        mesh=mesh,
        compiler_params=pltpu.CompilerParams(needs_layout_passes=False),
        out_type=jax.ShapeDtypeStruct((nsamp, CP), jnp.float32),
        scratch_types=[
            pltpu.VMEM((samp_w,), jnp.float32),
            pltpu.VMEM((samp_w,), jnp.float32),
            pltpu.VMEM((2 * 9, CH), jnp.int32),
            pltpu.VMEM((2 * 9 * CH,), jnp.float32),
            pltpu.VMEM((2 * 9, CH, CP), jnp.float32),
            pltpu.VMEM((2 * CH, CP), jnp.float32),
            pltpu.SemaphoreType.DMA,
            pltpu.SemaphoreType.DMA,
            pltpu.SemaphoreType.DMA,
            pltpu.SemaphoreType.DMA,
        ],
    )
    def k(xt_hbm, px_hbm, py_hbm, out_hbm,
          px_v, py_v, idx_v, w_v, rows_v, out_v, gs0, gs1, os0, os1):
        gsem = [gs0, gs1]
        osem = [os0, os1]
        wid = lax.axis_index("s") * 2 + lax.axis_index("c")
        base = wid * samp_w
        pltpu.sync_copy(px_hbm.at[pl.ds(base, samp_w)], px_v)
        pltpu.sync_copy(py_hbm.at[pl.ds(base, samp_w)], py_v)
        nbat = nsamp // N_
        bterm = (wid // (NW // nbat)) * (NX * NY)

        def stage(g, b):
            l0 = g * CH
            for t in range(CH // L):
                o = l0 + t * L
                cx = px_v[pl.ds(o, L)] * jnp.float32(NX - 1)
                cy = py_v[pl.ds(o, L)] * jnp.float32(NY - 1)
                rx = _rne(cx)
                ry = _rne(cy)
                wx = _axis_weights(rx, cx, jnp.float32(NX))
                wy = _axis_weights(ry, cy, jnp.float32(NX))
                ix = [jnp.clip(rx + jnp.float32(ofs), 0.0, jnp.float32(NX - 1))
                      .astype(jnp.int32) for ofs in IOFF]
                iy = [jnp.clip(ry + jnp.float32(ofs), 0.0, jnp.float32(NY - 1))
                      .astype(jnp.int32) for ofs in IOFF]
                for j in range(3):
                    rowj = ix[j] * NY + bterm
                    for kk in range(3):
                        tap = j * 3 + kk
                        idx_v[b * 9 + tap, pl.ds(t * L, L)] = rowj + iy[kk]
                        w_v[pl.ds((b * 9 + tap) * CH + t * L, L)] = wx[j] * wy[kk]
            for tap in range(9):
                pltpu.async_copy(xt_hbm.at[idx_v.at[b * 9 + tap]],
                                 rows_v.at[b * 9 + tap], gsem[b])

        def wait_gathers(b):
            for tap in range(9):
                pltpu.make_async_copy(xt_hbm.at[idx_v.at[b * 9 + tap]],
                                      rows_v.at[b * 9 + tap], gsem[b]).wait()

        def accumulate(g, b):
            def acc_body(s, _carry):
                sv = jnp.full((L,), s, jnp.int32)
                wb = [plsc.load_gather(w_v, [sv + (b * 9 + tap) * CH])
                      for tap in range(9)]
                for c6 in range(C_ // L):
                    a = wb[0] * rows_v[b * 9, s, pl.ds(c6 * L, L)]
                    for tap in range(1, 9):
                        a = a + wb[tap] * rows_v[b * 9 + tap, s, pl.ds(c6 * L, L)]
                    out_v[b * CH + s, pl.ds(c6 * L, L)] = a
                return 0
            lax.fori_loop(0, CH, acc_body, 0)
            pltpu.async_copy(out_v.at[pl.ds(b * CH, CH)],
                             out_hbm.at[pl.ds(base + g * CH, CH)], osem[b])

        def wait_out(g, b):
            pltpu.make_async_copy(out_v.at[pl.ds(b * CH, CH)],
                                  out_hbm.at[pl.ds(base + g * CH, CH)],
                                  osem[b]).wait()

        stage(0, 0)
        stage(1, 1)

        def body(i, _):
            for b in (0, 1):
                g = 2 * i + b
                wait_gathers(b)

                @pl.when(i > 0)
                def _():
                    wait_out(g - 2, b)

                accumulate(g, b)
                stage(g + 2, b)
            return 0

        lax.fori_loop(0, nchunk // 2 - 1, body, 0)

        for b in (0, 1):
            g = nchunk - 2 + b
            wait_gathers(b)
            wait_out(g - 2, b)
            accumulate(g, b)
            wait_out(g, b)

    return k


_sc_sample = _make_sc_kernel(B_ * N_)

NPIX = NX * NY
RB = 64


def _tc_transpose_body(x_ref, o_ref):
    zpad = jnp.zeros((NY, CP - C_), jnp.float32)
    for rr in range(RB):
        t = jnp.transpose(x_ref[0, :, rr, :], (1, 0))
        o_ref[pl.ds(rr * NY, NY), :] = jnp.concatenate([t, zpad], axis=1)


_tc_transpose = pl.pallas_call(
    _tc_transpose_body,
    grid=(B_, NX // RB),
    in_specs=[pl.BlockSpec((1, C_, RB, NY), lambda b, r: (b, 0, r, 0))],
    out_specs=pl.BlockSpec((RB * NY, CP),
                           lambda b, r: (b * (NX // RB) + r, 0)),
    out_shape=jax.ShapeDtypeStruct((B_ * NPIX, CP), jnp.float32),
)


def kernel(x, coords):
    b, c, nx, ny = x.shape
    n = coords.shape[1]
    assert (b, c, nx, ny) == (B_, C_, NX, NY) and n == N_
    xt = _tc_transpose(x)
    px = coords[:, :, 1].reshape(-1)
    py = coords[:, :, 0].reshape(-1)
    out = _sc_sample(xt, px, py)
    return out[:, :c].reshape(b, n, c).transpose(0, 2, 1)
